# Initial kernel scaffold; baseline (speedup 1.0000x reference)
import jax
import jax.numpy as jnp
from jax.experimental import pallas as pl


def kernel(inputs, emb_weight):
    raise NotImplementedError("write your pallas kernel here")


# speedup vs baseline: 8.0692x; 8.0692x over previous
import functools

import jax
import jax.numpy as jnp
from jax import lax
from jax.experimental import pallas as pl
from jax.experimental.pallas import tpu as pltpu
from jax.experimental.pallas import tpu_sc as plsc

_K = 8192
_D = 256
_N = 8192
_BR = 256
_R = _N // _BR
_BC = 2048
_C = _K // _BC
_COMMIT = 0.25


def _bf16_rtne(x):
    bits = jax.lax.bitcast_convert_type(x, jnp.uint32)
    lsb = (bits >> 16) & jnp.uint32(1)
    rounded = (bits + jnp.uint32(0x7FFF) + lsb) & jnp.uint32(0xFFFF0000)
    return jax.lax.bitcast_convert_type(rounded, jnp.float32)


def _vq_body(z_ref, emb_ref, z2_ref, e2_ref, idx_ref, enc_ref, loss_ref,
             perp_ref, counts_ref, dsum_ref):
    i = pl.program_id(0)
    z = z_ref[...]
    z2 = z2_ref[...]

    @pl.when(i == 0)
    def _init():
        counts_ref[...] = jnp.zeros_like(counts_ref)
        dsum_ref[0] = 0.0

    run_min = jnp.full((_BR, 1), jnp.inf, dtype=jnp.float32)
    run_true = jnp.zeros((_BR, 1), dtype=jnp.float32)
    run_idx = jnp.zeros((_BR, 1), dtype=jnp.int32)
    for j in range(_C):
        emb_c = emb_ref[j * _BC:(j + 1) * _BC, :]
        dot = lax.dot_general(z, emb_c, (((1,), (1,)), ((), ())),
                              preferred_element_type=jnp.float32)
        e2 = e2_ref[0:1, j * _BC:(j + 1) * _BC]
        dist = (z2 + e2) - 2.0 * dot
        lmin = jnp.min(dist, axis=1, keepdims=True)
        cols = lax.broadcasted_iota(jnp.int32, (_BR, _BC), 1) + j * _BC
        lidx = jnp.min(jnp.where(dist == lmin, cols, _K),
                       axis=1, keepdims=True)
        better = lmin < run_min
        run_min = jnp.where(better, _bf16_rtne(lmin), run_min)
        run_true = jnp.where(better, lmin, run_true)
        run_idx = jnp.where(better, lidx, run_idx)

    idx_ref[...] = run_idx
    for j in range(_C):
        cols = lax.broadcasted_iota(jnp.int32, (_BR, _BC), 1) + j * _BC
        encj = (cols == run_idx).astype(jnp.float32)
        enc_ref[:, j * _BC:(j + 1) * _BC] = encj
        counts_ref[0:1, j * _BC:(j + 1) * _BC] += jnp.sum(
            encj, axis=0, keepdims=True)
    dsum_ref[0] += jnp.sum(run_true)

    @pl.when(i == _R - 1)
    def _fin():
        loss = (1.0 + _COMMIT) * dsum_ref[0] / float(_N * _D)
        loss_ref[...] = jnp.reshape(loss, (1, 1))
        p = counts_ref[...] * (1.0 / _N)
        ent = -jnp.sum(p * jnp.log(p + 1e-10))
        perp_ref[...] = jnp.reshape(jnp.exp(ent), (1, 1))


def _vq_tc(flat, emb, z2, e2):
    return pl.pallas_call(
        _vq_body,
        grid=(_R,),
        in_specs=[
            pl.BlockSpec((_BR, _D), lambda i: (i, 0)),
            pl.BlockSpec((_K, _D), lambda i: (0, 0)),
            pl.BlockSpec((_BR, 1), lambda i: (i, 0)),
            pl.BlockSpec((1, _K), lambda i: (0, 0)),
        ],
        out_specs=[
            pl.BlockSpec((_BR, 1), lambda i: (i, 0)),
            pl.BlockSpec((_BR, _K), lambda i: (i, 0)),
            pl.BlockSpec((1, 1), lambda i: (0, 0)),
            pl.BlockSpec((1, 1), lambda i: (0, 0)),
        ],
        out_shape=[
            jax.ShapeDtypeStruct((_N, 1), jnp.int32),
            jax.ShapeDtypeStruct((_N, _K), jnp.float32),
            jax.ShapeDtypeStruct((1, 1), jnp.float32),
            jax.ShapeDtypeStruct((1, 1), jnp.float32),
        ],
        scratch_shapes=[
            pltpu.VMEM((1, _K), jnp.float32),
            pltpu.SMEM((1,), jnp.float32),
        ],
    )(flat, emb, z2, e2)


_GCH = 128


def _gather(emb, idx):
    info = plsc.get_sparse_core_info()
    nw = info.num_cores * info.num_subcores
    bpw = _N // nw
    nch = bpw // _GCH
    ncores = info.num_cores

    def body(emb_hbm, idx_hbm, out_hbm, idx_v, rows_v, sem):
        wid = lax.axis_index("s") * ncores + lax.axis_index("c")
        base = wid * bpw
        for c in range(nch):
            off = base + c * _GCH
            pltpu.sync_copy(idx_hbm.at[pl.ds(off, _GCH)], idx_v)
            pltpu.async_copy(emb_hbm.at[idx_v], rows_v, sem).wait()
            pltpu.sync_copy(rows_v, out_hbm.at[pl.ds(off, _GCH)])

    call = pl.kernel(
        body,
        mesh=plsc.VectorSubcoreMesh(core_axis_name="c", subcore_axis_name="s"),
        out_type=jax.ShapeDtypeStruct((_N, _D), jnp.float32),
        scratch_types=[
            pltpu.VMEM((_GCH,), jnp.int32),
            pltpu.VMEM((_GCH, _D), jnp.float32),
            pltpu.SemaphoreType.DMA,
        ],
    )
    return call(emb, idx)


def kernel(inputs, emb_weight):
    flat = inputs.reshape(_N, _D)
    z2 = jnp.sum(flat ** 2, axis=1, keepdims=True)
    e2 = jnp.sum(emb_weight ** 2, axis=1)[None, :]
    idx2, enc, loss, perp = _vq_tc(flat, emb_weight, z2, e2)
    q = _gather(emb_weight, idx2.reshape(_N))
    quantized = q.reshape(inputs.shape)
    return (loss[0, 0], quantized, perp[0, 0], enc)


# speedup vs baseline: 8.1337x; 1.0080x over previous
import jax
import jax.numpy as jnp
from jax import lax
from jax.experimental import pallas as pl
from jax.experimental.pallas import tpu as pltpu
from jax.experimental.pallas import tpu_sc as plsc

_K = 8192
_D = 256
_N = 8192
_BR = 256
_R = _N // _BR
_BC = 2048
_C = _K // _BC
_COMMIT = 0.25


def _bf16_rtne(x):
    bits = jax.lax.bitcast_convert_type(x, jnp.uint32)
    lsb = (bits >> 16) & jnp.uint32(1)
    rounded = (bits + jnp.uint32(0x7FFF) + lsb) & jnp.uint32(0xFFFF0000)
    return jax.lax.bitcast_convert_type(rounded, jnp.float32)


def _vq_body(z_ref, emb_ref, z2_ref, e2_ref, idx_ref, enc_ref, loss_ref,
             perp_ref, counts_ref, dsum_ref):
    i = pl.program_id(0)
    z = z_ref[...]
    z2 = z2_ref[...]

    @pl.when(i == 0)
    def _init():
        counts_ref[...] = jnp.zeros_like(counts_ref)
        dsum_ref[0] = 0.0

    run_min = jnp.full((_BR, 1), jnp.inf, dtype=jnp.float32)
    run_true = jnp.zeros((_BR, 1), dtype=jnp.float32)
    run_idx = jnp.zeros((_BR, 1), dtype=jnp.int32)
    for j in range(_C):
        emb_c = emb_ref[j * _BC:(j + 1) * _BC, :]
        dot = lax.dot_general(z, emb_c, (((1,), (1,)), ((), ())),
                              preferred_element_type=jnp.float32)
        e2 = e2_ref[0:1, j * _BC:(j + 1) * _BC]
        dist = (z2 + e2) - 2.0 * dot
        lmin = jnp.min(dist, axis=1, keepdims=True)
        cols = lax.broadcasted_iota(jnp.int32, (_BR, _BC), 1) + j * _BC
        lidx = jnp.min(jnp.where(dist == lmin, cols, _K),
                       axis=1, keepdims=True)
        better = lmin < run_min
        run_min = jnp.where(better, _bf16_rtne(lmin), run_min)
        run_true = jnp.where(better, lmin, run_true)
        run_idx = jnp.where(better, lidx, run_idx)

    idx_ref[...] = run_idx
    for j in range(_C):
        cols = lax.broadcasted_iota(jnp.int32, (_BR, _BC), 1) + j * _BC
        encj = (cols == run_idx).astype(jnp.float32)
        enc_ref[:, j * _BC:(j + 1) * _BC] = encj
        counts_ref[0:1, j * _BC:(j + 1) * _BC] += jnp.sum(
            encj, axis=0, keepdims=True)
    dsum_ref[0] += jnp.sum(run_true)

    @pl.when(i == _R - 1)
    def _fin():
        loss = (1.0 + _COMMIT) * dsum_ref[0] / float(_N * _D)
        loss_ref[...] = jnp.reshape(loss, (1, 1))
        p = counts_ref[...] * (1.0 / _N)
        ent = -jnp.sum(p * jnp.log(p + 1e-10))
        perp_ref[...] = jnp.reshape(jnp.exp(ent), (1, 1))


def _vq_tc(flat, emb, z2, e2):
    return pl.pallas_call(
        _vq_body,
        grid=(_R,),
        in_specs=[
            pl.BlockSpec((_BR, _D), lambda i: (i, 0)),
            pl.BlockSpec((_K, _D), lambda i: (0, 0)),
            pl.BlockSpec((_BR, 1), lambda i: (i, 0)),
            pl.BlockSpec((1, _K), lambda i: (0, 0)),
        ],
        out_specs=[
            pl.BlockSpec((_BR, 1), lambda i: (i, 0)),
            pl.BlockSpec((_BR, _K), lambda i: (i, 0)),
            pl.BlockSpec((1, 1), lambda i: (0, 0)),
            pl.BlockSpec((1, 1), lambda i: (0, 0)),
        ],
        out_shape=[
            jax.ShapeDtypeStruct((_N, 1), jnp.int32),
            jax.ShapeDtypeStruct((_N, _K), jnp.float32),
            jax.ShapeDtypeStruct((1, 1), jnp.float32),
            jax.ShapeDtypeStruct((1, 1), jnp.float32),
        ],
        scratch_shapes=[
            pltpu.VMEM((1, _K), jnp.float32),
            pltpu.SMEM((1,), jnp.float32),
        ],
    )(flat, emb, z2, e2)


_GCH = 128


def _gather(emb, idx):
    info = plsc.get_sparse_core_info()
    nw = info.num_cores * info.num_subcores
    bpw = _N // nw
    nch = bpw // _GCH
    ncores = info.num_cores

    def body(emb_hbm, idx_hbm, out_hbm, idx_v, rows_v, sem):
        wid = lax.axis_index("s") * ncores + lax.axis_index("c")
        base = wid * bpw
        for c in range(nch):
            off = base + c * _GCH
            pltpu.sync_copy(idx_hbm.at[pl.ds(off, _GCH)], idx_v)
            pltpu.async_copy(emb_hbm.at[idx_v], rows_v, sem).wait()
            pltpu.sync_copy(rows_v, out_hbm.at[pl.ds(off, _GCH)])

    call = pl.kernel(
        body,
        mesh=plsc.VectorSubcoreMesh(core_axis_name="c", subcore_axis_name="s"),
        out_type=jax.ShapeDtypeStruct((_N, _D), jnp.float32),
        scratch_types=[
            pltpu.VMEM((_GCH,), jnp.int32),
            pltpu.VMEM((_GCH, _D), jnp.float32),
            pltpu.SemaphoreType.DMA,
        ],
    )
    return call(emb, idx)


def kernel(inputs, emb_weight):
    flat = inputs.reshape(_N, _D)
    z2 = jnp.sum(flat ** 2, axis=1, keepdims=True)
    e2 = jnp.sum(emb_weight ** 2, axis=1)[None, :]
    idx2, enc, loss, perp = _vq_tc(flat, emb_weight, z2, e2)
    q = _gather(emb_weight, idx2.reshape(_N))
    quantized = q.reshape(inputs.shape)
    return (loss[0, 0], quantized, perp[0, 0], enc)
